# baseline (speedup 1.0000x reference)
import functools

import jax
import jax.numpy as jnp
import numpy as np
from jax.experimental import pallas as pl
from jax.experimental.pallas import tpu as pltpu

_K = 16
_N = 2048
_BR = 2048
_FINF = np.float32(np.inf)


def _main_kernel(at_ref, ar_ref, w1t_ref, s1_ref, t1_ref, w2t_ref, s2_ref,
                 t2_ref, w4t_ref, s4_ref, t4_ref, out_ref):
    rb = pl.program_id(1)
    xa = at_ref[0]
    xb = ar_ref[0]

    xa_c = [xa[c:c + 1, :] for c in range(3)]
    xb_c = [xb[:, c:c + 1] for c in range(3)]

    sqa = xa_c[0] * xa_c[0] + xa_c[1] * xa_c[1] + xa_c[2] * xa_c[2]
    sqb = xb_c[0] * xb_c[0] + xb_c[1] * xb_c[1] + xb_c[2] * xb_c[2]
    cross = jnp.dot(xb.astype(jnp.bfloat16), xa.astype(jnp.bfloat16),
                    preferred_element_type=jnp.float32)
    pd = (sqb - 2.0 * cross) + sqa

    iota = jax.lax.broadcasted_iota(jnp.int32, (_BR, _N), 1)
    bits = jax.lax.bitcast_convert_type(jnp.maximum(pd, 0.0), jnp.int32)
    qkey = ((bits & np.int32(-2048)) | iota) + np.int32(1 << 23)
    fkey = jax.lax.bitcast_convert_type(qkey, jnp.float32)
    for _ in range(_K):
        m = jnp.min(fkey, axis=1, keepdims=True)
        fkey = jnp.where(fkey == m, _FINF, fkey)
    msk = (fkey == _FINF).astype(jnp.float32)

    s1 = [jnp.sum(msk * xa_c[c], axis=1, keepdims=True) for c in range(3)]
    s2 = {}
    for c in range(3):
        for d_ in range(c, 3):
            s2[(c, d_)] = jnp.sum(msk * (xa_c[c] * xa_c[d_]), axis=1,
                                  keepdims=True)
    inv_k = np.float32(1.0 / _K)
    feat = list(xb_c)
    for c in range(3):
        for d_ in range(3):
            key = (c, d_) if c <= d_ else (d_, c)
            feat.append(s2[key] - s1[c] * s1[d_] * inv_k)

    z = jnp.zeros((_BR, 32), jnp.float32)
    for f in range(12):
        z = z + feat[f] * w1t_ref[f:f + 1, :]
    h = jnp.maximum(z * s1_ref[...] + t1_ref[...], 0.0)

    h = jnp.dot(h, w2t_ref[...], preferred_element_type=jnp.float32)
    h = jnp.maximum(h * s2_ref[...] + t2_ref[...], 0.0)

    h = jnp.dot(h, w4t_ref[...], preferred_element_type=jnp.float32)
    h = jnp.maximum(h * s4_ref[...] + t4_ref[...], 0.0)

    bm = jnp.max(h, axis=0, keepdims=True)

    @pl.when(rb == 0)
    def _init():
        out_ref[0] = bm

    @pl.when(rb != 0)
    def _acc():
        out_ref[0] = jnp.maximum(out_ref[0], bm)


def _head_kernel(p_ref, w1_ref, s1_ref, t1_ref, w2_ref, s2_ref, t2_ref,
                 w3_ref, s3_ref, t3_ref, w4_ref, b4_ref, w5_ref, b5_ref,
                 out_ref):
    h = p_ref[...]
    h = jnp.dot(h, w1_ref[...], preferred_element_type=jnp.float32)
    h = jnp.maximum(h * s1_ref[...] + t1_ref[...], 0.0)
    h = jnp.dot(h, w2_ref[...], preferred_element_type=jnp.float32)
    h = jnp.maximum(h * s2_ref[...] + t2_ref[...], 0.0)
    h = jnp.dot(h, w3_ref[...], preferred_element_type=jnp.float32)
    h = jnp.maximum(h * s3_ref[...] + t3_ref[...], 0.0)
    h = jnp.tanh(jnp.dot(h, w4_ref[...],
                         preferred_element_type=jnp.float32) + b4_ref[...])
    z = jnp.sum(h * w5_ref[...], axis=1, keepdims=True) + b5_ref[...]
    out_ref[...] = 1.0 / (1.0 + jnp.exp(-z))


@functools.partial(jax.jit, static_argnums=())
def kernel(data, W1, b1, g1, be1, W2, b2, g2, be2, W4, b4, g4, be4,
           fcW1, fcb1, fcg1, fcbe1, fcW2, fcb2, fcg2, fcbe2,
           fcW3, fcb3, fcg3, fcbe3, fcW4, fcb4, fcW5, fcb5):
    B, N, C = data.shape
    inv = np.float32(1.0 / np.sqrt(1.0 + 1e-5))

    def fold(g, be, b_):
        s = g * inv
        return s, b_ * s + be

    s1, t1 = fold(g1, be1, b1)
    s2, t2 = fold(g2, be2, b2)
    s4, t4 = fold(g4, be4, b4)
    fs1, ft1 = fold(fcg1, fcbe1, fcb1)
    fs2, ft2 = fold(fcg2, fcbe2, fcb2)
    fs3, ft3 = fold(fcg3, fcbe3, fcb3)

    dataT = jnp.transpose(data, (0, 2, 1))

    grid = (B, N // _BR)
    pooled = pl.pallas_call(
        _main_kernel,
        grid=grid,
        in_specs=[
            pl.BlockSpec((1, 3, N), lambda b, r: (b, 0, 0)),
            pl.BlockSpec((1, _BR, 3), lambda b, r: (b, r, 0)),
            pl.BlockSpec((12, 32), lambda b, r: (0, 0)),
            pl.BlockSpec((1, 32), lambda b, r: (0, 0)),
            pl.BlockSpec((1, 32), lambda b, r: (0, 0)),
            pl.BlockSpec((32, 32), lambda b, r: (0, 0)),
            pl.BlockSpec((1, 32), lambda b, r: (0, 0)),
            pl.BlockSpec((1, 32), lambda b, r: (0, 0)),
            pl.BlockSpec((32, 1024), lambda b, r: (0, 0)),
            pl.BlockSpec((1, 1024), lambda b, r: (0, 0)),
            pl.BlockSpec((1, 1024), lambda b, r: (0, 0)),
        ],
        out_specs=pl.BlockSpec((1, 1, 1024), lambda b, r: (b, 0, 0)),
        out_shape=jax.ShapeDtypeStruct((B, 1, 1024), jnp.float32),
    )(dataT, data, W1.T, s1[None, :], t1[None, :], W2.T, s2[None, :],
      t2[None, :], W4.T, s4[None, :], t4[None, :])
    pooled = pooled.reshape(B, 1024)

    out = pl.pallas_call(
        _head_kernel,
        out_shape=jax.ShapeDtypeStruct((B, 1), jnp.float32),
    )(pooled, fcW1.T, fs1[None, :], ft1[None, :], fcW2.T, fs2[None, :],
      ft2[None, :], fcW3.T, fs3[None, :], ft3[None, :], fcW4.T,
      fcb4[None, :], fcW5, fcb5[None, :])
    return out


# speedup vs baseline: 1.2527x; 1.2527x over previous
import functools

import jax
import jax.numpy as jnp
import numpy as np
from jax.experimental import pallas as pl


_K = 16
_N = 2048
_BR = 1024
_FINF = np.float32(np.inf)


def _main_kernel(at_ref, ar_ref, w1t_ref, s1_ref, t1_ref, w2t_ref, s2_ref,
                 t2_ref, w4t_ref, s4_ref, t4_ref, out_ref):
    rb = pl.program_id(1)
    xa = at_ref[0]
    xb = ar_ref[0]

    xa_c = [xa[c:c + 1, :] for c in range(3)]
    xb_c = [xb[:, c:c + 1] for c in range(3)]

    sqa = xa_c[0] * xa_c[0] + xa_c[1] * xa_c[1] + xa_c[2] * xa_c[2]
    sqb = xb_c[0] * xb_c[0] + xb_c[1] * xb_c[1] + xb_c[2] * xb_c[2]
    cross = jnp.dot(xb.astype(jnp.bfloat16), xa.astype(jnp.bfloat16),
                    preferred_element_type=jnp.float32)
    pd = (sqb - 2.0 * cross) + sqa

    iota = jax.lax.broadcasted_iota(jnp.int32, (_BR, _N), 1)
    bits = jax.lax.bitcast_convert_type(jnp.maximum(pd, 0.0), jnp.int32)
    qkey = ((bits & np.int32(-2048)) | iota) + np.int32(1 << 23)
    fkey = jax.lax.bitcast_convert_type(qkey, jnp.float32)
    for _ in range(_K):
        m = jnp.min(fkey, axis=1, keepdims=True)
        fkey = jnp.where(fkey == m, _FINF, fkey)
    msk = (fkey == _FINF).astype(jnp.float32)

    s1 = [jnp.sum(msk * xa_c[c], axis=1, keepdims=True) for c in range(3)]
    s2 = {}
    for c in range(3):
        for d_ in range(c, 3):
            s2[(c, d_)] = jnp.sum(msk * (xa_c[c] * xa_c[d_]), axis=1,
                                  keepdims=True)
    inv_k = np.float32(1.0 / _K)
    feat = list(xb_c)
    for c in range(3):
        for d_ in range(3):
            key = (c, d_) if c <= d_ else (d_, c)
            feat.append(s2[key] - s1[c] * s1[d_] * inv_k)

    z = jnp.zeros((_BR, 32), jnp.float32)
    for f in range(12):
        z = z + feat[f] * w1t_ref[f:f + 1, :]
    h = jnp.maximum(z * s1_ref[...] + t1_ref[...], 0.0)

    h = jnp.dot(h, w2t_ref[...], preferred_element_type=jnp.float32)
    h = jnp.maximum(h * s2_ref[...] + t2_ref[...], 0.0)

    h = jnp.dot(h, w4t_ref[...], preferred_element_type=jnp.float32)
    h = jnp.maximum(h * s4_ref[...] + t4_ref[...], 0.0)

    bm = jnp.max(h, axis=0, keepdims=True)

    @pl.when(rb == 0)
    def _init():
        out_ref[0] = bm

    @pl.when(rb != 0)
    def _acc():
        out_ref[0] = jnp.maximum(out_ref[0], bm)


def _head_kernel(p_ref, w1_ref, s1_ref, t1_ref, w2_ref, s2_ref, t2_ref,
                 w3_ref, s3_ref, t3_ref, w4_ref, b4_ref, w5_ref, b5_ref,
                 out_ref):
    h = p_ref[...]
    h = jnp.dot(h, w1_ref[...], preferred_element_type=jnp.float32)
    h = jnp.maximum(h * s1_ref[...] + t1_ref[...], 0.0)
    h = jnp.dot(h, w2_ref[...], preferred_element_type=jnp.float32)
    h = jnp.maximum(h * s2_ref[...] + t2_ref[...], 0.0)
    h = jnp.dot(h, w3_ref[...], preferred_element_type=jnp.float32)
    h = jnp.maximum(h * s3_ref[...] + t3_ref[...], 0.0)
    h = jnp.tanh(jnp.dot(h, w4_ref[...],
                         preferred_element_type=jnp.float32) + b4_ref[...])
    z = jnp.sum(h * w5_ref[...], axis=1, keepdims=True) + b5_ref[...]
    out_ref[...] = 1.0 / (1.0 + jnp.exp(-z))


@functools.partial(jax.jit, static_argnums=())
def kernel(data, W1, b1, g1, be1, W2, b2, g2, be2, W4, b4, g4, be4,
           fcW1, fcb1, fcg1, fcbe1, fcW2, fcb2, fcg2, fcbe2,
           fcW3, fcb3, fcg3, fcbe3, fcW4, fcb4, fcW5, fcb5):
    B, N, C = data.shape
    inv = np.float32(1.0 / np.sqrt(1.0 + 1e-5))

    def fold(g, be, b_):
        s = g * inv
        return s, b_ * s + be

    s1, t1 = fold(g1, be1, b1)
    s2, t2 = fold(g2, be2, b2)
    s4, t4 = fold(g4, be4, b4)
    fs1, ft1 = fold(fcg1, fcbe1, fcb1)
    fs2, ft2 = fold(fcg2, fcbe2, fcb2)
    fs3, ft3 = fold(fcg3, fcbe3, fcb3)

    dataT = jnp.transpose(data, (0, 2, 1))

    grid = (B, N // _BR)
    pooled = pl.pallas_call(
        _main_kernel,
        grid=grid,
        in_specs=[
            pl.BlockSpec((1, 3, N), lambda b, r: (b, 0, 0)),
            pl.BlockSpec((1, _BR, 3), lambda b, r: (b, r, 0)),
            pl.BlockSpec((12, 32), lambda b, r: (0, 0)),
            pl.BlockSpec((1, 32), lambda b, r: (0, 0)),
            pl.BlockSpec((1, 32), lambda b, r: (0, 0)),
            pl.BlockSpec((32, 32), lambda b, r: (0, 0)),
            pl.BlockSpec((1, 32), lambda b, r: (0, 0)),
            pl.BlockSpec((1, 32), lambda b, r: (0, 0)),
            pl.BlockSpec((32, 1024), lambda b, r: (0, 0)),
            pl.BlockSpec((1, 1024), lambda b, r: (0, 0)),
            pl.BlockSpec((1, 1024), lambda b, r: (0, 0)),
        ],
        out_specs=pl.BlockSpec((1, 1, 1024), lambda b, r: (b, 0, 0)),
        out_shape=jax.ShapeDtypeStruct((B, 1, 1024), jnp.float32),
    )(dataT, data, W1.T, s1[None, :], t1[None, :], W2.T, s2[None, :],
      t2[None, :], W4.T, s4[None, :], t4[None, :])
    pooled = pooled.reshape(B, 1024)

    out = pl.pallas_call(
        _head_kernel,
        out_shape=jax.ShapeDtypeStruct((B, 1), jnp.float32),
    )(pooled, fcW1.T, fs1[None, :], ft1[None, :], fcW2.T, fs2[None, :],
      ft2[None, :], fcW3.T, fs3[None, :], ft3[None, :], fcW4.T,
      fcb4[None, :], fcW5, fcb5[None, :])
    return out
